# baseline (speedup 1.0000x reference)
import functools

import jax
import jax.numpy as jnp
from jax.experimental import pallas as pl
from jax.experimental.pallas import tpu as pltpu

N = 50000
D = 256
G = 128
K = 16
DOUT = 128

BLK = 5000
NB = N // BLK


def _body(x_ref, batch_ref, ev_ref, filt_ref, gw1_ref, gb1_ref,
          mw1_ref, mb1_ref, mw2_ref, mb2_ref, out_ref, acc_ref, cnt_ref):
    i = pl.program_id(0)

    @pl.when(i == 0)
    def _init():
        acc_ref[...] = jnp.zeros_like(acc_ref)
        cnt_ref[...] = jnp.zeros_like(cnt_ref)

    xb = x_ref[...]
    h = jnp.dot(xb, gw1_ref[...], preferred_element_type=jnp.float32)
    h = h + gb1_ref[...]
    x = xb.astype(jnp.float32)
    xt = (h * x) / (1.0 + jnp.exp2(h * (-1.4426950408889634)))

    w = jnp.sum((ev_ref[...] * filt_ref[...]).astype(jnp.float32),
                axis=1, keepdims=True)

    brow = batch_ref[0]
    glab = jax.lax.broadcasted_iota(jnp.int32, (G, BLK), 0)
    oht = (jnp.broadcast_to(brow, (G, BLK)) == glab).astype(jnp.float32)

    acc_ref[:, :D] += jnp.dot(oht, xt * w, preferred_element_type=jnp.float32)
    acc_ref[:, D:] += jnp.dot(oht, xt, preferred_element_type=jnp.float32)
    ones = jnp.full((BLK, 8), 1.0, dtype=jnp.float32)
    cnt_ref[:, :8] += jnp.dot(oht, ones, preferred_element_type=jnp.float32)

    @pl.when(i == NB - 1)
    def _finish():
        acc = acc_ref[...]
        spec = acc[:, :D]
        sums = acc[:, D:]
        counts = cnt_ref[:, 0:1]
        pooled = sums / jnp.maximum(counts, 1.0) + spec
        h2 = jnp.dot(pooled, mw1_ref[...], preferred_element_type=jnp.float32)
        h2 = jnp.maximum(h2 + mb1_ref[...], 0.0)
        out = jnp.dot(h2, mw2_ref[...], preferred_element_type=jnp.float32)
        out_ref[...] = out + mb2_ref[...]


@jax.jit
def kernel(x, batch, eigvec, filt, glu_w1, glu_b1, mlp_w1, mlp_b1,
           mlp_w2, mlp_b2):
    batch_f = batch.astype(jnp.int32).reshape(NB, 1, BLK)
    x = x.astype(jnp.bfloat16)
    eigvec = eigvec.astype(jnp.bfloat16)
    glu_w1 = glu_w1.astype(jnp.bfloat16)
    filt_r = filt.reshape(1, K).astype(jnp.bfloat16)
    gb1 = glu_b1.reshape(1, D)
    mb1 = mlp_b1.reshape(1, D)
    mb2 = mlp_b2.reshape(1, DOUT)

    grid = (NB,)
    out = pl.pallas_call(
        _body,
        grid=grid,
        in_specs=[
            pl.BlockSpec((BLK, D), lambda i: (i, 0)),
            pl.BlockSpec((1, 1, BLK), lambda i: (i, 0, 0)),
            pl.BlockSpec((BLK, K), lambda i: (i, 0)),
            pl.BlockSpec((1, K), lambda i: (0, 0)),
            pl.BlockSpec((D, D), lambda i: (0, 0)),
            pl.BlockSpec((1, D), lambda i: (0, 0)),
            pl.BlockSpec((D, D), lambda i: (0, 0)),
            pl.BlockSpec((1, D), lambda i: (0, 0)),
            pl.BlockSpec((D, DOUT), lambda i: (0, 0)),
            pl.BlockSpec((1, DOUT), lambda i: (0, 0)),
        ],
        out_specs=pl.BlockSpec((G, DOUT), lambda i: (0, 0)),
        out_shape=jax.ShapeDtypeStruct((G, DOUT), jnp.float32),
        scratch_shapes=[
            pltpu.VMEM((G, 2 * D), jnp.float32),
            pltpu.VMEM((G, 128), jnp.float32),
        ],
        compiler_params=pltpu.CompilerParams(
            dimension_semantics=("arbitrary",),
        ),
    )(x, batch_f, eigvec, filt_r, glu_w1, gb1, mlp_w1, mb1, mlp_w2, mb2)
    return out


# speedup vs baseline: 1.4347x; 1.4347x over previous
import functools

import jax
import jax.numpy as jnp
from jax.experimental import pallas as pl
from jax.experimental.pallas import tpu as pltpu

N = 50000
D = 256
G = 128
K = 16
DOUT = 128

BLK = 5000
NB = N // BLK


def _body(x_ref, batch_ref, ev_ref, filt_ref, gw1_ref, gb1_ref,
          mw1_ref, mb1_ref, mw2_ref, mb2_ref, out_ref, acc_ref, cnt_ref,
          oht_ref):
    i = pl.program_id(0)

    @pl.when(i == 0)
    def _init():
        acc_ref[...] = jnp.zeros_like(acc_ref)
        cnt_ref[...] = jnp.zeros_like(cnt_ref)

    x = x_ref[...]
    h = jnp.dot(x, gw1_ref[...], preferred_element_type=jnp.float32)
    h = h + gb1_ref[...]
    xt = (h * x) / (1.0 + jnp.exp2(h * (-1.4426950408889634)))

    w = jnp.sum(ev_ref[...] * filt_ref[...], axis=1, keepdims=True)

    brow = batch_ref[0]
    glab = jax.lax.broadcasted_iota(jnp.int32, (G, BLK), 0)
    oht_ref[...] = (jnp.broadcast_to(brow, (G, BLK)) == glab).astype(
        jnp.float32)
    oht = oht_ref[...]

    acc_ref[:, :D] += jnp.dot(oht, xt * w, preferred_element_type=jnp.float32)
    acc_ref[:, D:] += jnp.dot(oht_ref[...], xt,
                              preferred_element_type=jnp.float32)
    ones = jnp.full((BLK, 8), 1.0, dtype=jnp.float32)
    cnt_ref[:, :8] += jnp.dot(oht_ref[...], ones,
                              preferred_element_type=jnp.float32)

    @pl.when(i == NB - 1)
    def _finish():
        acc = acc_ref[...]
        spec = acc[:, :D]
        sums = acc[:, D:]
        counts = cnt_ref[:, 0:1]
        pooled = sums / jnp.maximum(counts, 1.0) + spec
        h2 = jnp.dot(pooled, mw1_ref[...], preferred_element_type=jnp.float32)
        h2 = jnp.maximum(h2 + mb1_ref[...], 0.0)
        out = jnp.dot(h2, mw2_ref[...], preferred_element_type=jnp.float32)
        out_ref[...] = out + mb2_ref[...]


@jax.jit
def kernel(x, batch, eigvec, filt, glu_w1, glu_b1, mlp_w1, mlp_b1,
           mlp_w2, mlp_b2):
    batch_f = batch.astype(jnp.int32).reshape(NB, 1, BLK)
    filt_r = filt.reshape(1, K)
    gb1 = glu_b1.reshape(1, D)
    mb1 = mlp_b1.reshape(1, D)
    mb2 = mlp_b2.reshape(1, DOUT)

    grid = (NB,)
    out = pl.pallas_call(
        _body,
        grid=grid,
        in_specs=[
            pl.BlockSpec((BLK, D), lambda i: (i, 0)),
            pl.BlockSpec((1, 1, BLK), lambda i: (i, 0, 0)),
            pl.BlockSpec((BLK, K), lambda i: (i, 0)),
            pl.BlockSpec((1, K), lambda i: (0, 0)),
            pl.BlockSpec((D, D), lambda i: (0, 0)),
            pl.BlockSpec((1, D), lambda i: (0, 0)),
            pl.BlockSpec((D, D), lambda i: (0, 0)),
            pl.BlockSpec((1, D), lambda i: (0, 0)),
            pl.BlockSpec((D, DOUT), lambda i: (0, 0)),
            pl.BlockSpec((1, DOUT), lambda i: (0, 0)),
        ],
        out_specs=pl.BlockSpec((G, DOUT), lambda i: (0, 0)),
        out_shape=jax.ShapeDtypeStruct((G, DOUT), jnp.float32),
        scratch_shapes=[
            pltpu.VMEM((G, 2 * D), jnp.float32),
            pltpu.VMEM((G, 128), jnp.float32),
            pltpu.VMEM((G, BLK), jnp.float32),
        ],
        compiler_params=pltpu.CompilerParams(
            dimension_semantics=("arbitrary",),
        ),
    )(x, batch_f, eigvec, filt_r, glu_w1, gb1, mlp_w1, mb1, mlp_w2, mb2)
    return out


# speedup vs baseline: 1.4855x; 1.0354x over previous
import functools

import jax
import jax.numpy as jnp
from jax.experimental import pallas as pl
from jax.experimental.pallas import tpu as pltpu

N = 50000
D = 256
G = 128
K = 16
DOUT = 128

BLK = 10000
NB = N // BLK


def _body(x_ref, batch_ref, ev_ref, filt_ref, gw1_ref, gb1_ref,
          mw1_ref, mb1_ref, mw2_ref, mb2_ref, out_ref, acc_ref, cnt_ref):
    i = pl.program_id(0)

    @pl.when(i == 0)
    def _init():
        acc_ref[...] = jnp.zeros_like(acc_ref)
        cnt_ref[...] = jnp.zeros_like(cnt_ref)

    x = x_ref[...]
    h = jnp.dot(x, gw1_ref[...], preferred_element_type=jnp.float32)
    h = h + gb1_ref[...]
    xt = (h * x) / (1.0 + jnp.exp(-h))

    w = jnp.sum(ev_ref[...] * filt_ref[...], axis=1, keepdims=True)

    brow = batch_ref[0]
    glab = jax.lax.broadcasted_iota(jnp.int32, (G, BLK), 0)
    oht = (jnp.broadcast_to(brow, (G, BLK)) == glab).astype(jnp.float32)

    acc_ref[:, :D] += jnp.dot(oht, xt * w, preferred_element_type=jnp.float32)
    acc_ref[:, D:] += jnp.dot(oht, xt, preferred_element_type=jnp.float32)
    cnt_ref[...] += jnp.broadcast_to(
        jnp.sum(oht, axis=1, keepdims=True), (G, 128))

    @pl.when(i == NB - 1)
    def _finish():
        acc = acc_ref[...]
        spec = acc[:, :D]
        sums = acc[:, D:]
        counts = cnt_ref[:, 0:1]
        pooled = sums / jnp.maximum(counts, 1.0) + spec
        h2 = jnp.dot(pooled, mw1_ref[...], preferred_element_type=jnp.float32)
        h2 = jnp.maximum(h2 + mb1_ref[...], 0.0)
        out = jnp.dot(h2, mw2_ref[...], preferred_element_type=jnp.float32)
        out_ref[...] = out + mb2_ref[...]


@jax.jit
def kernel(x, batch, eigvec, filt, glu_w1, glu_b1, mlp_w1, mlp_b1,
           mlp_w2, mlp_b2):
    batch_f = batch.astype(jnp.int32).reshape(NB, 1, BLK)
    filt_r = filt.reshape(1, K)
    gb1 = glu_b1.reshape(1, D)
    mb1 = mlp_b1.reshape(1, D)
    mb2 = mlp_b2.reshape(1, DOUT)

    grid = (NB,)
    out = pl.pallas_call(
        _body,
        grid=grid,
        in_specs=[
            pl.BlockSpec((BLK, D), lambda i: (i, 0)),
            pl.BlockSpec((1, 1, BLK), lambda i: (i, 0, 0)),
            pl.BlockSpec((BLK, K), lambda i: (i, 0)),
            pl.BlockSpec((1, K), lambda i: (0, 0)),
            pl.BlockSpec((D, D), lambda i: (0, 0)),
            pl.BlockSpec((1, D), lambda i: (0, 0)),
            pl.BlockSpec((D, D), lambda i: (0, 0)),
            pl.BlockSpec((1, D), lambda i: (0, 0)),
            pl.BlockSpec((D, DOUT), lambda i: (0, 0)),
            pl.BlockSpec((1, DOUT), lambda i: (0, 0)),
        ],
        out_specs=pl.BlockSpec((G, DOUT), lambda i: (0, 0)),
        out_shape=jax.ShapeDtypeStruct((G, DOUT), jnp.float32),
        scratch_shapes=[
            pltpu.VMEM((G, 2 * D), jnp.float32),
            pltpu.VMEM((G, 128), jnp.float32),
        ],
        compiler_params=pltpu.CompilerParams(
            dimension_semantics=("arbitrary",),
        ),
    )(x, batch_f, eigvec, filt_r, glu_w1, gb1, mlp_w1, mb1, mlp_w2, mb2)
    return out
